# baseline (speedup 1.0000x reference)
import functools

import jax
import jax.numpy as jnp
from jax.experimental import pallas as pl
from jax.experimental.pallas import tpu as pltpu

_PAD = 72


def _conv_nhwc_kernel(x_ref, w_ref, b_ref, o_ref, s_ref, xp_ref, *, h, w, kk):
    cin = x_ref.shape[3]
    cout = o_ref.shape[3]
    m = h * w
    r = kk // 2
    tm = min(1024, m)

    xf = x_ref[0].reshape(m, cin)
    for t in range(0, m, tm):
        xp_ref[_PAD + t:_PAD + t + tm, :] = xf[t:t + tm, :].astype(jnp.bfloat16)
    xp_ref[0:_PAD, :] = jnp.zeros((_PAD, cin), jnp.bfloat16)
    xp_ref[_PAD + m:, :] = jnp.zeros((_PAD, cin), jnp.bfloat16)

    for t in range(0, m, tm):
        pos = t + jax.lax.broadcasted_iota(jnp.int32, (tm, 1), 0)
        col = jax.lax.rem(pos, w)
        for dy in range(kk):
            dr = dy - r
            for dx in range(kk):
                dc = dx - r
                s = dr * w + dc
                sl = xp_ref[_PAD + t + s:_PAD + t + s + tm, :]
                if dc < 0:
                    sl = jnp.where(col >= -dc, sl, jnp.bfloat16(0.0))
                elif dc > 0:
                    sl = jnp.where(col < w - dc, sl, jnp.bfloat16(0.0))
                lo = (dy * kk + dx) * cin
                s_ref[t:t + tm, lo:lo + cin] = sl

        a = jnp.dot(s_ref[t:t + tm, :], w_ref[...],
                    preferred_element_type=jnp.float32)
        y = a + b_ref[...]
        sig = pl.reciprocal(1.0 + jnp.exp(-y), approx=True)
        o_ref[0, t // w:(t + tm) // w] = (y * sig).reshape(tm // w, w, cout)


@functools.partial(jax.jit, static_argnames=("kernel_size", "eps"))
def _conv_bn_swish(x_nchw, weight, gamma, beta, running_mean,
                   running_var, *, kernel_size, eps=1e-5):
    n, cin, h, w = x_nchw.shape
    cout = weight.shape[0]
    kk = kernel_size
    m = h * w

    inv_std = gamma.astype(jnp.float32) / jnp.sqrt(
        running_var.astype(jnp.float32) + eps)
    bias = beta.astype(jnp.float32) - running_mean.astype(jnp.float32) * inv_std

    w_prep = jnp.transpose(weight.astype(jnp.float32) * inv_std[:, None, None, None],
                           (2, 3, 1, 0)).reshape(kk * kk * cin, cout).astype(jnp.bfloat16)
    b_prep = bias.reshape(1, cout)

    x_nhwc = jnp.transpose(x_nchw, (0, 2, 3, 1))

    kern = functools.partial(_conv_nhwc_kernel, h=h, w=w, kk=kk)

    out = pl.pallas_call(
        kern,
        out_shape=jax.ShapeDtypeStruct((n, h, w, cout), jnp.float32),
        grid=(n,),
        in_specs=[
            pl.BlockSpec((1, h, w, cin), lambda i: (i, 0, 0, 0)),
            pl.BlockSpec((kk * kk * cin, cout), lambda i: (0, 0)),
            pl.BlockSpec((1, cout), lambda i: (0, 0)),
        ],
        out_specs=pl.BlockSpec((1, h, w, cout), lambda i: (i, 0, 0, 0)),
        scratch_shapes=[
            pltpu.VMEM((m, kk * kk * cin), jnp.bfloat16),
            pltpu.VMEM((_PAD + m + _PAD, cin), jnp.bfloat16),
        ],
        compiler_params=pltpu.CompilerParams(
            dimension_semantics=("parallel",),
            vmem_limit_bytes=64 << 20,
        ),
        cost_estimate=pl.CostEstimate(
            flops=2 * n * m * kk * kk * cin * cout,
            transcendentals=n * m * cout,
            bytes_accessed=n * cin * m * 4 + n * cout * m * 4
            + kk * kk * cin * cout * 2),
    )(x_nhwc, w_prep, b_prep)

    return jnp.transpose(out, (0, 3, 1, 2))


def kernel(x_nchw, weight, gamma, beta, running_mean, running_var):
    return _conv_bn_swish(x_nchw, weight, gamma, beta, running_mean,
                          running_var, kernel_size=3)


# speedup vs baseline: 1.7116x; 1.7116x over previous
import functools

import jax
import jax.numpy as jnp
from jax.experimental import pallas as pl
from jax.experimental.pallas import tpu as pltpu


def _conv_nhwc_kernel(x_ref, w_ref, b_ref, o_ref, s_ref, *, h, w, kk):
    cin = x_ref.shape[3]
    cout = o_ref.shape[3]
    m = h * w
    r = kk // 2

    xb = x_ref[0].reshape(m, cin).astype(jnp.bfloat16)

    pos = jax.lax.broadcasted_iota(jnp.int32, (m, 1), 0)
    col = jax.lax.rem(pos, w)

    variants = []
    for dx in range(kk):
        dc = dx - r
        if dc == 0:
            variants.append(xb)
            continue
        xs = jnp.roll(xb, -dc, axis=0)
        valid = col >= -dc if dc < 0 else col < w - dc
        variants.append(jnp.where(valid, xs, jnp.bfloat16(0.0)))

    for dy in range(kk):
        dr = dy - r
        a = max(0, -dr * w)
        b = m - max(0, dr * w)
        for dx in range(kk):
            lo = (dy * kk + dx) * cin
            s_ref[a:b, lo:lo + cin] = variants[dx][a + dr * w:b + dr * w, :]
            if a > 0:
                s_ref[0:a, lo:lo + cin] = jnp.zeros((a, cin), jnp.bfloat16)
            if b < m:
                s_ref[b:m, lo:lo + cin] = jnp.zeros((m - b, cin), jnp.bfloat16)

    tm = min(1024, m)
    for t in range(0, m, tm):
        a = jnp.dot(s_ref[t:t + tm, :], w_ref[...],
                    preferred_element_type=jnp.float32)
        y = a + b_ref[...]
        sig = pl.reciprocal(1.0 + jnp.exp(-y), approx=True)
        o_ref[0, t // w:(t + tm) // w] = (y * sig).reshape(tm // w, w, cout)


@functools.partial(jax.jit, static_argnames=("kernel_size", "eps"))
def _conv_bn_swish(x_nchw, weight, gamma, beta, running_mean,
                   running_var, *, kernel_size, eps=1e-5):
    n, cin, h, w = x_nchw.shape
    cout = weight.shape[0]
    kk = kernel_size
    m = h * w

    inv_std = gamma.astype(jnp.float32) / jnp.sqrt(
        running_var.astype(jnp.float32) + eps)
    bias = beta.astype(jnp.float32) - running_mean.astype(jnp.float32) * inv_std

    w_prep = jnp.transpose(weight.astype(jnp.float32) * inv_std[:, None, None, None],
                           (2, 3, 1, 0)).reshape(kk * kk * cin, cout).astype(jnp.bfloat16)
    b_prep = bias.reshape(1, cout)

    x_nhwc = jnp.transpose(x_nchw, (0, 2, 3, 1))

    kern = functools.partial(_conv_nhwc_kernel, h=h, w=w, kk=kk)

    out = pl.pallas_call(
        kern,
        out_shape=jax.ShapeDtypeStruct((n, h, w, cout), jnp.float32),
        grid=(n,),
        in_specs=[
            pl.BlockSpec((1, h, w, cin), lambda i: (i, 0, 0, 0)),
            pl.BlockSpec((kk * kk * cin, cout), lambda i: (0, 0)),
            pl.BlockSpec((1, cout), lambda i: (0, 0)),
        ],
        out_specs=pl.BlockSpec((1, h, w, cout), lambda i: (i, 0, 0, 0)),
        scratch_shapes=[pltpu.VMEM((m, kk * kk * cin), jnp.bfloat16)],
        compiler_params=pltpu.CompilerParams(
            dimension_semantics=("parallel",),
            vmem_limit_bytes=64 << 20,
        ),
        cost_estimate=pl.CostEstimate(
            flops=2 * n * m * kk * kk * cin * cout,
            transcendentals=n * m * cout,
            bytes_accessed=n * cin * m * 4 + n * cout * m * 4
            + kk * kk * cin * cout * 2),
    )(x_nhwc, w_prep, b_prep)

    return jnp.transpose(out, (0, 3, 1, 2))


def kernel(x_nchw, weight, gamma, beta, running_mean, running_var):
    return _conv_bn_swish(x_nchw, weight, gamma, beta, running_mean,
                          running_var, kernel_size=3)


# speedup vs baseline: 1.9321x; 1.1289x over previous
import functools

import jax
import jax.numpy as jnp
from jax.experimental import pallas as pl
from jax.experimental.pallas import tpu as pltpu


def _conv_nhwc_kernel(x_ref, w_ref, b_ref, o_ref, s_ref, *, h, w, kk):
    cin = x_ref.shape[3]
    cout = o_ref.shape[3]
    m = h * w
    r = kk // 2
    pad = r * w

    xb = x_ref[0].reshape(m, cin).astype(jnp.bfloat16)

    pos = jax.lax.broadcasted_iota(jnp.int32, (m, 1), 0)
    col = jax.lax.rem(pos, w)

    variants = []
    for dx in range(kk):
        dc = dx - r
        if dc == 0:
            variants.append(xb)
            continue
        xs = jnp.roll(xb, -dc, axis=0)
        valid = col >= -dc if dc < 0 else col < w - dc
        variants.append(jnp.where(valid, xs, jnp.bfloat16(0.0)))

    s_ref[0:pad, :] = jnp.zeros((pad, kk * cin), jnp.bfloat16)
    s_ref[pad:pad + m, :] = jnp.concatenate(variants, axis=1)
    s_ref[pad + m:, :] = jnp.zeros((pad, kk * cin), jnp.bfloat16)

    tm = min(1024, m)
    for t in range(0, m, tm):
        xk = jnp.concatenate(
            [s_ref[dy * w + t:dy * w + t + tm, :] for dy in range(kk)],
            axis=1)
        a = jnp.dot(xk, w_ref[...], preferred_element_type=jnp.float32)
        y = a + b_ref[...]
        sig = pl.reciprocal(1.0 + jnp.exp(-y), approx=True)
        o_ref[0, t // w:(t + tm) // w] = (y * sig).reshape(tm // w, w, cout)


@functools.partial(jax.jit, static_argnames=("kernel_size", "eps"))
def _conv_bn_swish(x_nchw, weight, gamma, beta, running_mean,
                   running_var, *, kernel_size, eps=1e-5):
    n, cin, h, w = x_nchw.shape
    cout = weight.shape[0]
    kk = kernel_size
    m = h * w
    pad = (kk // 2) * w

    inv_std = gamma.astype(jnp.float32) / jnp.sqrt(
        running_var.astype(jnp.float32) + eps)
    bias = beta.astype(jnp.float32) - running_mean.astype(jnp.float32) * inv_std

    w_prep = jnp.transpose(weight.astype(jnp.float32) * inv_std[:, None, None, None],
                           (2, 3, 1, 0)).reshape(kk * kk * cin, cout).astype(jnp.bfloat16)
    b_prep = bias.reshape(1, cout)

    x_nhwc = jnp.transpose(x_nchw, (0, 2, 3, 1))

    kern = functools.partial(_conv_nhwc_kernel, h=h, w=w, kk=kk)

    out = pl.pallas_call(
        kern,
        out_shape=jax.ShapeDtypeStruct((n, h, w, cout), jnp.float32),
        grid=(n,),
        in_specs=[
            pl.BlockSpec((1, h, w, cin), lambda i: (i, 0, 0, 0)),
            pl.BlockSpec((kk * kk * cin, cout), lambda i: (0, 0)),
            pl.BlockSpec((1, cout), lambda i: (0, 0)),
        ],
        out_specs=pl.BlockSpec((1, h, w, cout), lambda i: (i, 0, 0, 0)),
        scratch_shapes=[pltpu.VMEM((pad + m + pad, kk * cin), jnp.bfloat16)],
        compiler_params=pltpu.CompilerParams(
            dimension_semantics=("parallel",),
            vmem_limit_bytes=64 << 20,
        ),
        cost_estimate=pl.CostEstimate(
            flops=2 * n * m * kk * kk * cin * cout,
            transcendentals=n * m * cout,
            bytes_accessed=n * cin * m * 4 + n * cout * m * 4
            + kk * kk * cin * cout * 2),
    )(x_nhwc, w_prep, b_prep)

    return jnp.transpose(out, (0, 3, 1, 2))


def kernel(x_nchw, weight, gamma, beta, running_mean, running_var):
    return _conv_bn_swish(x_nchw, weight, gamma, beta, running_mean,
                          running_var, kernel_size=3)


# speedup vs baseline: 2.0394x; 1.0555x over previous
import functools

import jax
import jax.numpy as jnp
from jax.experimental import pallas as pl
from jax.experimental.pallas import tpu as pltpu


def _conv_nhwc_kernel(x_ref, w_ref, b_ref, o_ref, s_ref, *, h, w, kk):
    cin = x_ref.shape[3]
    cout = o_ref.shape[3]
    m = h * w
    r = kk // 2
    pad = r * w

    xb = x_ref[0].reshape(m, cin).astype(jnp.bfloat16)

    pos = jax.lax.broadcasted_iota(jnp.int32, (m, 1), 0)
    col = jax.lax.rem(pos, w)

    variants = []
    for dx in range(kk):
        dc = dx - r
        if dc == 0:
            variants.append(xb)
            continue
        xs = jnp.roll(xb, -dc, axis=0)
        valid = col >= -dc if dc < 0 else col < w - dc
        variants.append(jnp.where(valid, xs, jnp.bfloat16(0.0)))

    s_ref[0:pad, :] = jnp.zeros((pad, kk * cin), jnp.bfloat16)
    s_ref[pad:pad + m, :] = jnp.concatenate(variants, axis=1)
    s_ref[pad + m:, :] = jnp.zeros((pad, kk * cin), jnp.bfloat16)

    tm = min(2048, m)
    for t in range(0, m, tm):
        xk = jnp.concatenate(
            [s_ref[dy * w + t:dy * w + t + tm, :] for dy in range(kk)],
            axis=1)
        a = jnp.dot(xk, w_ref[...], preferred_element_type=jnp.float32)
        y = a + b_ref[...]
        sig = pl.reciprocal(1.0 + jnp.exp(-y), approx=True)
        o_ref[0, t // w:(t + tm) // w] = (y * sig).reshape(tm // w, w, cout)


@functools.partial(jax.jit, static_argnames=("kernel_size", "eps"))
def _conv_bn_swish(x_nchw, weight, gamma, beta, running_mean,
                   running_var, *, kernel_size, eps=1e-5):
    n, cin, h, w = x_nchw.shape
    cout = weight.shape[0]
    kk = kernel_size
    m = h * w
    pad = (kk // 2) * w

    inv_std = gamma.astype(jnp.float32) / jnp.sqrt(
        running_var.astype(jnp.float32) + eps)
    bias = beta.astype(jnp.float32) - running_mean.astype(jnp.float32) * inv_std

    w_prep = jnp.transpose(weight.astype(jnp.float32) * inv_std[:, None, None, None],
                           (2, 3, 1, 0)).reshape(kk * kk * cin, cout).astype(jnp.bfloat16)
    b_prep = bias.reshape(1, cout)

    x_nhwc = jnp.transpose(x_nchw, (0, 2, 3, 1))

    kern = functools.partial(_conv_nhwc_kernel, h=h, w=w, kk=kk)

    out = pl.pallas_call(
        kern,
        out_shape=jax.ShapeDtypeStruct((n, h, w, cout), jnp.float32),
        grid=(n,),
        in_specs=[
            pl.BlockSpec((1, h, w, cin), lambda i: (i, 0, 0, 0)),
            pl.BlockSpec((kk * kk * cin, cout), lambda i: (0, 0)),
            pl.BlockSpec((1, cout), lambda i: (0, 0)),
        ],
        out_specs=pl.BlockSpec((1, h, w, cout), lambda i: (i, 0, 0, 0)),
        scratch_shapes=[pltpu.VMEM((pad + m + pad, kk * cin), jnp.bfloat16)],
        compiler_params=pltpu.CompilerParams(
            dimension_semantics=("parallel",),
            vmem_limit_bytes=64 << 20,
        ),
        cost_estimate=pl.CostEstimate(
            flops=2 * n * m * kk * kk * cin * cout,
            transcendentals=n * m * cout,
            bytes_accessed=n * cin * m * 4 + n * cout * m * 4
            + kk * kk * cin * cout * 2),
    )(x_nhwc, w_prep, b_prep)

    return jnp.transpose(out, (0, 3, 1, 2))


def kernel(x_nchw, weight, gamma, beta, running_mean, running_var):
    return _conv_bn_swish(x_nchw, weight, gamma, beta, running_mean,
                          running_var, kernel_size=3)
